# Initial kernel scaffold; baseline (speedup 1.0000x reference)
import jax
import jax.numpy as jnp
from jax.experimental import pallas as pl


def kernel(fine_scale_h_d, prolongation_map_fine_to_coarse):
    raise NotImplementedError("write your pallas kernel here")


# speedup vs baseline: 6.6115x; 6.6115x over previous
import functools

import jax
import jax.numpy as jnp
from jax import lax
from jax.experimental import pallas as pl
from jax.experimental.pallas import tpu as pltpu
from jax.experimental.pallas import tpu_sc as plsc

N = 320000
D = 128
S = 10000
NC, NS = 2, 16
NW = NC * NS
ROWS_PER_W = N // NW
B = 80
CHUNKS = ROWS_PER_W // B
S_PAD = 10240
SEG_PER_TILE = S_PAD // NS
L = 16


def _sc_body(fine_hbm, ids_hbm, zsum_hbm,
             sum_out, cnt_out, idx_v, row_v, cnt_v, acc_s):
    c = lax.axis_index("c")
    s = lax.axis_index("s")
    w = c * NS + s

    pltpu.sync_copy(zsum_hbm, acc_s.at[pl.ds(s * SEG_PER_TILE, SEG_PER_TILE), :])

    def zero_cnt(j, carry):
        cnt_v[pl.ds(j * L, L)] = jnp.zeros((L,), jnp.float32)
        return carry

    lax.fori_loop(0, S_PAD // L, zero_cnt, 0)
    plsc.subcore_barrier()

    ones = jnp.ones((L,), jnp.float32)

    def chunk(g, carry):
        base = w * ROWS_PER_W + g * B
        pltpu.sync_copy(ids_hbm.at[pl.ds(base, B)], idx_v)
        pltpu.sync_copy(fine_hbm.at[pl.ds(base, B), :], row_v)
        pltpu.sync_copy(row_v, acc_s.at[idx_v], add=True)
        for j in range(B // L):
            idx16 = idx_v[pl.ds(j * L, L)]
            plsc.addupdate_scatter(cnt_v, [idx16], ones)
        return carry

    lax.fori_loop(0, CHUNKS, chunk, 0)

    plsc.subcore_barrier()
    pltpu.sync_copy(acc_s.at[pl.ds(s * SEG_PER_TILE, SEG_PER_TILE), :],
                    sum_out.at[c, pl.ds(s * SEG_PER_TILE, SEG_PER_TILE), :])
    pltpu.sync_copy(cnt_v, cnt_out.at[w])


_sc_segment_sums = functools.partial(
    pl.kernel,
    out_type=(
        jax.ShapeDtypeStruct((NC, S_PAD, D), jnp.float32),
        jax.ShapeDtypeStruct((NW, S_PAD), jnp.float32),
    ),
    mesh=plsc.VectorSubcoreMesh(
        core_axis_name="c", subcore_axis_name="s", num_cores=NC, num_subcores=NS
    ),
    compiler_params=pltpu.CompilerParams(needs_layout_passes=False),
    scratch_types=[
        pltpu.VMEM((B,), jnp.int32),
        pltpu.VMEM((B, D), jnp.float32),
        pltpu.VMEM((S_PAD,), jnp.float32),
        pltpu.VMEM_SHARED((S_PAD, D), jnp.float32),
    ],
)(_sc_body)


def _combine_body(sum_ref, cnt_ref, out_ref):
    total = sum_ref[0] + sum_ref[1]
    cnt = jnp.sum(cnt_ref[...], axis=0)[:, None]
    out_ref[...] = jnp.where(cnt > 0, total / jnp.maximum(cnt, 1.0),
                             jnp.zeros_like(total))


_COMBINE_BS = 1024


def _combine(sums, cnts):
    grid = pl.cdiv(S, _COMBINE_BS)
    return pl.pallas_call(
        _combine_body,
        grid=(grid,),
        in_specs=[
            pl.BlockSpec((NC, _COMBINE_BS, D), lambda i: (0, i, 0)),
            pl.BlockSpec((NW, _COMBINE_BS), lambda i: (0, i)),
        ],
        out_specs=pl.BlockSpec((_COMBINE_BS, D), lambda i: (i, 0)),
        out_shape=jax.ShapeDtypeStruct((S, D), jnp.float32),
    )(sums, cnts)


def kernel(fine_scale_h_d, prolongation_map_fine_to_coarse):
    zsum = jnp.zeros((SEG_PER_TILE, D), jnp.float32)
    sums, cnts = _sc_segment_sums(
        fine_scale_h_d, prolongation_map_fine_to_coarse, zsum
    )
    return _combine(sums, cnts)


# speedup vs baseline: 10.5359x; 1.5936x over previous
import functools

import jax
import jax.numpy as jnp
from jax import lax
from jax.experimental import pallas as pl
from jax.experimental.pallas import tpu as pltpu
from jax.experimental.pallas import tpu_sc as plsc

N = 320000
D = 128
S = 10000
NC, NS = 2, 16
NW = NC * NS
ROWS_PER_W = N // NW
B = 80
CHUNKS = ROWS_PER_W // B
S_PAD = 10240
SEG_PER_TILE = S_PAD // NS
L = 16


def _sc_body(fine_hbm, ids3_hbm,
             sum_out, cnt_out,
             idx_big, buf0, buf1, cnt_v,
             sem_l0, sem_l1, sem_s0, sem_s1, acc_s):
    c = lax.axis_index("c")
    s = lax.axis_index("s")
    w = c * NS + s

    def zero_buf(i, carry):
        buf0[i // (D // L), pl.ds((i % (D // L)) * L, L)] = jnp.zeros((L,), jnp.float32)
        return carry

    lax.fori_loop(0, B * D // L, zero_buf, 0)

    def zero_cnt(j, carry):
        cnt_v[pl.ds(j * L, L)] = jnp.zeros((L,), jnp.float32)
        return carry

    lax.fori_loop(0, S_PAD // L, zero_cnt, 0)

    for q in range(SEG_PER_TILE // B):
        pltpu.sync_copy(buf0, acc_s.at[pl.ds(s * SEG_PER_TILE + q * B, B), :])
    plsc.subcore_barrier()

    pltpu.sync_copy(ids3_hbm.at[w], idx_big)

    ones = jnp.ones((L,), jnp.float32)

    def row_slice(g):
        return fine_hbm.at[pl.ds(w * ROWS_PER_W + g * B, B), :]

    def counts(g):
        for j in range(B // L):
            plsc.addupdate_scatter(cnt_v, [idx_big[g, pl.ds(j * L, L)]], ones)

    pltpu.async_copy(row_slice(0), buf0, sem_l0).wait()
    pltpu.async_copy(row_slice(1), buf1, sem_l1)
    pltpu.async_copy(buf0, acc_s.at[idx_big.at[0]], sem_s0, add=True)
    counts(0)
    pltpu.make_async_copy(row_slice(1), buf1, sem_l1).wait()
    pltpu.make_async_copy(buf0, acc_s.at[idx_big.at[0]], sem_s0).wait()
    pltpu.async_copy(row_slice(2), buf0, sem_l0)
    pltpu.async_copy(buf1, acc_s.at[idx_big.at[1]], sem_s1, add=True)
    counts(1)

    def body(k, carry):
        g0 = 2 * k
        g1 = 2 * k + 1
        pltpu.make_async_copy(row_slice(g0), buf0, sem_l0).wait()
        pltpu.make_async_copy(buf1, acc_s.at[idx_big.at[g1 - 2]], sem_s1).wait()
        pltpu.async_copy(row_slice(g1), buf1, sem_l1)
        pltpu.async_copy(buf0, acc_s.at[idx_big.at[g0]], sem_s0, add=True)
        counts(g0)
        pltpu.make_async_copy(row_slice(g1), buf1, sem_l1).wait()
        pltpu.make_async_copy(buf0, acc_s.at[idx_big.at[g0]], sem_s0).wait()
        pltpu.async_copy(row_slice(g0 + 2), buf0, sem_l0)
        pltpu.async_copy(buf1, acc_s.at[idx_big.at[g1]], sem_s1, add=True)
        counts(g1)
        return carry

    lax.fori_loop(1, (CHUNKS - 1) // 2, body, 0)

    g_last = CHUNKS - 1
    pltpu.make_async_copy(row_slice(g_last), buf0, sem_l0).wait()
    pltpu.make_async_copy(buf1, acc_s.at[idx_big.at[g_last - 1]], sem_s1).wait()
    pltpu.async_copy(buf0, acc_s.at[idx_big.at[g_last]], sem_s0, add=True)
    counts(g_last)
    pltpu.make_async_copy(buf0, acc_s.at[idx_big.at[g_last]], sem_s0).wait()

    plsc.subcore_barrier()
    pltpu.sync_copy(acc_s.at[pl.ds(s * SEG_PER_TILE, SEG_PER_TILE), :],
                    sum_out.at[c, pl.ds(s * SEG_PER_TILE, SEG_PER_TILE), :])
    pltpu.sync_copy(cnt_v, cnt_out.at[w])


_sc_segment_sums = functools.partial(
    pl.kernel,
    out_type=(
        jax.ShapeDtypeStruct((NC, S_PAD, D), jnp.float32),
        jax.ShapeDtypeStruct((NW, S_PAD), jnp.float32),
    ),
    mesh=plsc.VectorSubcoreMesh(
        core_axis_name="c", subcore_axis_name="s", num_cores=NC, num_subcores=NS
    ),
    compiler_params=pltpu.CompilerParams(needs_layout_passes=False),
    scratch_types=[
        pltpu.VMEM((CHUNKS, B), jnp.int32),
        pltpu.VMEM((B, D), jnp.float32),
        pltpu.VMEM((B, D), jnp.float32),
        pltpu.VMEM((S_PAD,), jnp.float32),
        pltpu.SemaphoreType.DMA,
        pltpu.SemaphoreType.DMA,
        pltpu.SemaphoreType.DMA,
        pltpu.SemaphoreType.DMA,
        pltpu.VMEM_SHARED((S_PAD, D), jnp.float32),
    ],
)(_sc_body)


def _combine_body(sum_ref, cnt_ref, out_ref):
    total = sum_ref[0] + sum_ref[1]
    cnt = jnp.sum(cnt_ref[...], axis=0)[:, None]
    out_ref[...] = jnp.where(cnt > 0, total / jnp.maximum(cnt, 1.0),
                             jnp.zeros_like(total))


_COMBINE_BS = 1024


def _combine(sums, cnts):
    grid = pl.cdiv(S, _COMBINE_BS)
    return pl.pallas_call(
        _combine_body,
        grid=(grid,),
        in_specs=[
            pl.BlockSpec((NC, _COMBINE_BS, D), lambda i: (0, i, 0)),
            pl.BlockSpec((NW, _COMBINE_BS), lambda i: (0, i)),
        ],
        out_specs=pl.BlockSpec((_COMBINE_BS, D), lambda i: (i, 0)),
        out_shape=jax.ShapeDtypeStruct((S, D), jnp.float32),
    )(sums, cnts)


def kernel(fine_scale_h_d, prolongation_map_fine_to_coarse):
    ids3 = prolongation_map_fine_to_coarse.reshape(NW, CHUNKS, B)
    sums, cnts = _sc_segment_sums(fine_scale_h_d, ids3)
    return _combine(sums, cnts)


# speedup vs baseline: 14.9527x; 1.4192x over previous
import functools

import jax
import jax.numpy as jnp
from jax import lax
from jax.experimental import pallas as pl
from jax.experimental.pallas import tpu as pltpu
from jax.experimental.pallas import tpu_sc as plsc

N = 320000
D = 128
S = 10000
NC, NS = 2, 16
NW = NC * NS
ROWS_PER_W = N // NW
B = 80
CHUNKS = ROWS_PER_W // B
S_PAD = 10240
SEG_PER_TILE = S_PAD // NS
L = 16


def _sc_body(fine_hbm, ids_hbm,
             sum_out, cnt_out,
             idxs, buf0, buf1, buf2, cnt_v,
             sem_l0, sem_l1, sem_l2,
             sem_s0, sem_s1, sem_s2, acc_s):
    c = lax.axis_index("c")
    s = lax.axis_index("s")
    w = c * NS + s
    bufs = (buf0, buf1, buf2)
    sem_l = (sem_l0, sem_l1, sem_l2)
    sem_s = (sem_s0, sem_s1, sem_s2)

    def zero_buf(i, carry):
        buf0[i // (D // L), pl.ds((i % (D // L)) * L, L)] = jnp.zeros((L,), jnp.float32)
        return carry

    lax.fori_loop(0, B * D // L, zero_buf, 0)

    def zero_cnt(j, carry):
        cnt_v[pl.ds(j * L, L)] = jnp.zeros((L,), jnp.float32)
        return carry

    lax.fori_loop(0, S_PAD // L, zero_cnt, 0)

    for q in range(SEG_PER_TILE // B):
        pltpu.sync_copy(buf0, acc_s.at[pl.ds(s * SEG_PER_TILE + q * B, B), :])
    plsc.subcore_barrier()

    ones = jnp.ones((L,), jnp.float32)

    def row_slice(g):
        return fine_hbm.at[pl.ds(w * ROWS_PER_W + g * B, B), :]

    def id_slice(g):
        return ids_hbm.at[pl.ds(w * ROWS_PER_W + g * B, B)]

    def fire_loads(g, b):
        pltpu.async_copy(row_slice(g), bufs[b], sem_l[b])
        pltpu.async_copy(id_slice(g), idxs.at[b], sem_l[b])

    def wait_loads(g, b):
        pltpu.make_async_copy(row_slice(g), bufs[b], sem_l[b]).wait()
        pltpu.make_async_copy(id_slice(g), idxs.at[b], sem_l[b]).wait()

    def counts(b):
        for j in range(B // L):
            plsc.addupdate_scatter(cnt_v, [idxs[b, pl.ds(j * L, L)]], ones)

    def stage(g, b):
        bp = (b + 2) % 3
        wait_loads(g, b)
        pltpu.async_copy(bufs[b], acc_s.at[idxs.at[b]], sem_s[b], add=True)
        counts(b)
        pltpu.make_async_copy(bufs[bp], acc_s.at[idxs.at[bp]], sem_s[bp]).wait()
        fire_loads(g + 2, bp)

    fire_loads(0, 0)
    fire_loads(1, 1)
    wait_loads(0, 0)
    pltpu.async_copy(buf0, acc_s.at[idxs.at[0]], sem_s0, add=True)
    counts(0)
    fire_loads(2, 2)

    def body(k, carry):
        g0 = 3 * k + 1
        stage(g0, 1)
        stage(g0 + 1, 2)
        stage(g0 + 2, 0)
        return carry

    lax.fori_loop(0, (CHUNKS - 5) // 3, body, 0)

    stage(CHUNKS - 4, 1)
    stage(CHUNKS - 3, 2)

    g = CHUNKS - 2
    wait_loads(g, 0)
    pltpu.async_copy(buf0, acc_s.at[idxs.at[0]], sem_s0, add=True)
    counts(0)
    pltpu.make_async_copy(buf2, acc_s.at[idxs.at[2]], sem_s2).wait()
    g = CHUNKS - 1
    wait_loads(g, 1)
    pltpu.async_copy(buf1, acc_s.at[idxs.at[1]], sem_s1, add=True)
    counts(1)
    pltpu.make_async_copy(buf0, acc_s.at[idxs.at[0]], sem_s0).wait()
    pltpu.make_async_copy(buf1, acc_s.at[idxs.at[1]], sem_s1).wait()

    plsc.subcore_barrier()
    pltpu.sync_copy(acc_s.at[pl.ds(s * SEG_PER_TILE, SEG_PER_TILE), :],
                    sum_out.at[c, pl.ds(s * SEG_PER_TILE, SEG_PER_TILE), :])
    pltpu.sync_copy(cnt_v, cnt_out.at[w])


_sc_segment_sums = functools.partial(
    pl.kernel,
    out_type=(
        jax.ShapeDtypeStruct((NC, S_PAD, D), jnp.float32),
        jax.ShapeDtypeStruct((NW, S_PAD), jnp.float32),
    ),
    mesh=plsc.VectorSubcoreMesh(
        core_axis_name="c", subcore_axis_name="s", num_cores=NC, num_subcores=NS
    ),
    compiler_params=pltpu.CompilerParams(needs_layout_passes=False),
    scratch_types=[
        pltpu.VMEM((3, B), jnp.int32),
        pltpu.VMEM((B, D), jnp.float32),
        pltpu.VMEM((B, D), jnp.float32),
        pltpu.VMEM((B, D), jnp.float32),
        pltpu.VMEM((S_PAD,), jnp.float32),
        pltpu.SemaphoreType.DMA,
        pltpu.SemaphoreType.DMA,
        pltpu.SemaphoreType.DMA,
        pltpu.SemaphoreType.DMA,
        pltpu.SemaphoreType.DMA,
        pltpu.SemaphoreType.DMA,
        pltpu.VMEM_SHARED((S_PAD, D), jnp.float32),
    ],
)(_sc_body)


def _combine_body(sum_ref, cnt_ref, out_ref):
    total = sum_ref[0] + sum_ref[1]
    cnt = jnp.sum(cnt_ref[...], axis=0)[:, None]
    out_ref[...] = jnp.where(cnt > 0, total / jnp.maximum(cnt, 1.0),
                             jnp.zeros_like(total))


_COMBINE_BS = 1024


def _combine(sums, cnts):
    grid = pl.cdiv(S, _COMBINE_BS)
    return pl.pallas_call(
        _combine_body,
        grid=(grid,),
        in_specs=[
            pl.BlockSpec((NC, _COMBINE_BS, D), lambda i: (0, i, 0)),
            pl.BlockSpec((NW, _COMBINE_BS), lambda i: (0, i)),
        ],
        out_specs=pl.BlockSpec((_COMBINE_BS, D), lambda i: (i, 0)),
        out_shape=jax.ShapeDtypeStruct((S, D), jnp.float32),
    )(sums, cnts)


def kernel(fine_scale_h_d, prolongation_map_fine_to_coarse):
    sums, cnts = _sc_segment_sums(fine_scale_h_d, prolongation_map_fine_to_coarse)
    return _combine(sums, cnts)


# speedup vs baseline: 15.3335x; 1.0255x over previous
import functools

import jax
import jax.numpy as jnp
from jax import lax
from jax.experimental import pallas as pl
from jax.experimental.pallas import tpu as pltpu
from jax.experimental.pallas import tpu_sc as plsc

N = 320000
D = 128
S = 10000
NC, NS = 2, 16
NW = NC * NS
ROWS_PER_W = N // NW
B = 80
CHUNKS = ROWS_PER_W // B
S_PAD = 10240
SEG_PER_TILE = S_PAD // NS
L = 16


def _sc_body(fine_hbm, ids_hbm,
             sum_out, cnt_out,
             idxs, buf0, buf1, buf2, cnt_v,
             sem_l0, sem_l1, sem_l2,
             sem_s0, sem_s1, sem_s2, acc_s):
    c = lax.axis_index("c")
    s = lax.axis_index("s")
    w = c * NS + s
    bufs = (buf0, buf1, buf2)
    sem_l = (sem_l0, sem_l1, sem_l2)
    sem_s = (sem_s0, sem_s1, sem_s2)

    def zero_buf(i, carry):
        buf0[i // (D // L), pl.ds((i % (D // L)) * L, L)] = jnp.zeros((L,), jnp.float32)
        return carry

    lax.fori_loop(0, B * D // L, zero_buf, 0)

    for q in range(SEG_PER_TILE // B):
        pltpu.async_copy(buf0, acc_s.at[pl.ds(s * SEG_PER_TILE + q * B, B), :],
                         sem_s0)

    def zero_cnt(j, carry):
        cnt_v[pl.ds(j * L, L)] = jnp.zeros((L,), jnp.float32)
        return carry

    lax.fori_loop(0, S_PAD // L, zero_cnt, 0)

    for q in range(SEG_PER_TILE // B):
        pltpu.make_async_copy(buf0, acc_s.at[pl.ds(s * SEG_PER_TILE + q * B, B), :],
                              sem_s0).wait()
    plsc.subcore_barrier()

    ones = jnp.ones((L,), jnp.float32)

    def row_slice(g):
        return fine_hbm.at[pl.ds(w * ROWS_PER_W + g * B, B), :]

    def id_slice(g):
        return ids_hbm.at[pl.ds(w * ROWS_PER_W + g * B, B)]

    def fire_loads(g, b):
        pltpu.async_copy(row_slice(g), bufs[b], sem_l[b])
        pltpu.async_copy(id_slice(g), idxs.at[b], sem_l[b])

    def wait_loads(g, b):
        pltpu.make_async_copy(row_slice(g), bufs[b], sem_l[b]).wait()
        pltpu.make_async_copy(id_slice(g), idxs.at[b], sem_l[b]).wait()

    def counts(b):
        for j in range(B // L):
            plsc.addupdate_scatter(cnt_v, [idxs[b, pl.ds(j * L, L)]], ones)

    def stage(g, b):
        bp = (b + 2) % 3
        wait_loads(g, b)
        pltpu.async_copy(bufs[b], acc_s.at[idxs.at[b]], sem_s[b], add=True)
        counts(b)
        pltpu.make_async_copy(bufs[bp], acc_s.at[idxs.at[bp]], sem_s[bp]).wait()
        fire_loads(g + 2, bp)

    fire_loads(0, 0)
    fire_loads(1, 1)
    wait_loads(0, 0)
    pltpu.async_copy(buf0, acc_s.at[idxs.at[0]], sem_s0, add=True)
    counts(0)
    fire_loads(2, 2)

    def body(k, carry):
        g0 = 3 * k + 1
        stage(g0, 1)
        stage(g0 + 1, 2)
        stage(g0 + 2, 0)
        return carry

    lax.fori_loop(0, (CHUNKS - 5) // 3, body, 0)

    stage(CHUNKS - 4, 1)
    stage(CHUNKS - 3, 2)

    g = CHUNKS - 2
    wait_loads(g, 0)
    pltpu.async_copy(buf0, acc_s.at[idxs.at[0]], sem_s0, add=True)
    counts(0)
    pltpu.make_async_copy(buf2, acc_s.at[idxs.at[2]], sem_s2).wait()
    g = CHUNKS - 1
    wait_loads(g, 1)
    pltpu.async_copy(buf1, acc_s.at[idxs.at[1]], sem_s1, add=True)
    counts(1)
    pltpu.make_async_copy(buf0, acc_s.at[idxs.at[0]], sem_s0).wait()
    pltpu.make_async_copy(buf1, acc_s.at[idxs.at[1]], sem_s1).wait()

    plsc.subcore_barrier()
    pltpu.sync_copy(acc_s.at[pl.ds(s * SEG_PER_TILE, SEG_PER_TILE), :],
                    sum_out.at[c, pl.ds(s * SEG_PER_TILE, SEG_PER_TILE), :])
    pltpu.sync_copy(cnt_v, cnt_out.at[w])


_sc_segment_sums = functools.partial(
    pl.kernel,
    out_type=(
        jax.ShapeDtypeStruct((NC, S_PAD, D), jnp.float32),
        jax.ShapeDtypeStruct((NW, S_PAD), jnp.float32),
    ),
    mesh=plsc.VectorSubcoreMesh(
        core_axis_name="c", subcore_axis_name="s", num_cores=NC, num_subcores=NS
    ),
    compiler_params=pltpu.CompilerParams(needs_layout_passes=False),
    scratch_types=[
        pltpu.VMEM((3, B), jnp.int32),
        pltpu.VMEM((B, D), jnp.float32),
        pltpu.VMEM((B, D), jnp.float32),
        pltpu.VMEM((B, D), jnp.float32),
        pltpu.VMEM((S_PAD,), jnp.float32),
        pltpu.SemaphoreType.DMA,
        pltpu.SemaphoreType.DMA,
        pltpu.SemaphoreType.DMA,
        pltpu.SemaphoreType.DMA,
        pltpu.SemaphoreType.DMA,
        pltpu.SemaphoreType.DMA,
        pltpu.VMEM_SHARED((S_PAD, D), jnp.float32),
    ],
)(_sc_body)


def _combine_body(sum_ref, cnt_ref, out_ref):
    total = sum_ref[0] + sum_ref[1]
    cnt = jnp.sum(cnt_ref[...], axis=0)[:, None]
    out_ref[...] = jnp.where(cnt > 0, total / jnp.maximum(cnt, 1.0),
                             jnp.zeros_like(total))


_COMBINE_BS = 1024


def _combine(sums, cnts):
    grid = pl.cdiv(S, _COMBINE_BS)
    return pl.pallas_call(
        _combine_body,
        grid=(grid,),
        in_specs=[
            pl.BlockSpec((NC, _COMBINE_BS, D), lambda i: (0, i, 0)),
            pl.BlockSpec((NW, _COMBINE_BS), lambda i: (0, i)),
        ],
        out_specs=pl.BlockSpec((_COMBINE_BS, D), lambda i: (i, 0)),
        out_shape=jax.ShapeDtypeStruct((S, D), jnp.float32),
    )(sums, cnts)


def kernel(fine_scale_h_d, prolongation_map_fine_to_coarse):
    sums, cnts = _sc_segment_sums(fine_scale_h_d, prolongation_map_fine_to_coarse)
    return _combine(sums, cnts)


# speedup vs baseline: 15.6412x; 1.0201x over previous
import functools

import jax
import jax.numpy as jnp
from jax import lax
from jax.experimental import pallas as pl
from jax.experimental.pallas import tpu as pltpu
from jax.experimental.pallas import tpu_sc as plsc

N = 320000
D = 128
S = 10000
NC, NS = 2, 16
NW = NC * NS
ROWS_PER_W = N // NW
B = 80
CHUNKS = ROWS_PER_W // B
S_PAD = 10240
SEG_PER_TILE = S_PAD // NS
L = 16


def _sc_body(fine_hbm, ids_hbm,
             sum_out, cnt_out,
             idxs, buf0, buf1, buf2, cnt_v,
             sem_l0, sem_l1, sem_l2,
             sem_s0, sem_s1, sem_s2, acc_s):
    c = lax.axis_index("c")
    s = lax.axis_index("s")
    w = c * NS + s
    bufs = (buf0, buf1, buf2)
    sem_l = (sem_l0, sem_l1, sem_l2)
    sem_s = (sem_s0, sem_s1, sem_s2)

    def zero_buf(i, carry):
        buf0[i // (D // L), pl.ds((i % (D // L)) * L, L)] = jnp.zeros((L,), jnp.float32)
        return carry

    lax.fori_loop(0, B * D // L, zero_buf, 0)

    for q in range(SEG_PER_TILE // B):
        pltpu.async_copy(buf0, acc_s.at[pl.ds(s * SEG_PER_TILE + q * B, B), :],
                         sem_s0)

    def zero_cnt(j, carry):
        cnt_v[pl.ds(j * L, L)] = jnp.zeros((L,), jnp.float32)
        return carry

    lax.fori_loop(0, S_PAD // L, zero_cnt, 0)

    for q in range(SEG_PER_TILE // B):
        pltpu.make_async_copy(buf0, acc_s.at[pl.ds(s * SEG_PER_TILE + q * B, B), :],
                              sem_s0).wait()
    plsc.subcore_barrier()

    ones = jnp.ones((L,), jnp.float32)

    def row_slice(g):
        return fine_hbm.at[pl.ds(w * ROWS_PER_W + g * B, B), :]

    def id_slice(g):
        return ids_hbm.at[pl.ds(w * ROWS_PER_W + g * B, B)]

    def fire_loads(g, b):
        pltpu.async_copy(row_slice(g), bufs[b], sem_l[b])
        pltpu.async_copy(id_slice(g), idxs.at[b], sem_l[b])

    def wait_loads(g, b):
        pltpu.make_async_copy(row_slice(g), bufs[b], sem_l[b]).wait()
        pltpu.make_async_copy(id_slice(g), idxs.at[b], sem_l[b]).wait()

    def counts(b):
        for j in range(B // L):
            plsc.addupdate_scatter(cnt_v, [idxs[b, pl.ds(j * L, L)]], ones)

    def stage(g, b):
        bp = (b + 2) % 3
        wait_loads(g, b)
        pltpu.async_copy(bufs[b], acc_s.at[idxs.at[b]], sem_s[b], add=True)
        counts(b)
        pltpu.make_async_copy(bufs[bp], acc_s.at[idxs.at[bp]], sem_s[bp]).wait()
        fire_loads(g + 2, bp)

    fire_loads(0, 0)
    fire_loads(1, 1)
    wait_loads(0, 0)
    pltpu.async_copy(buf0, acc_s.at[idxs.at[0]], sem_s0, add=True)
    counts(0)
    fire_loads(2, 2)

    def body(k, carry):
        g0 = 3 * k + 1
        stage(g0, 1)
        stage(g0 + 1, 2)
        stage(g0 + 2, 0)
        return carry

    lax.fori_loop(0, (CHUNKS - 5) // 3, body, 0)

    stage(CHUNKS - 4, 1)
    stage(CHUNKS - 3, 2)

    g = CHUNKS - 2
    wait_loads(g, 0)
    pltpu.async_copy(buf0, acc_s.at[idxs.at[0]], sem_s0, add=True)
    counts(0)
    pltpu.make_async_copy(buf2, acc_s.at[idxs.at[2]], sem_s2).wait()
    g = CHUNKS - 1
    wait_loads(g, 1)
    pltpu.async_copy(buf1, acc_s.at[idxs.at[1]], sem_s1, add=True)
    counts(1)
    pltpu.make_async_copy(buf0, acc_s.at[idxs.at[0]], sem_s0).wait()
    pltpu.make_async_copy(buf1, acc_s.at[idxs.at[1]], sem_s1).wait()

    plsc.subcore_barrier()
    pltpu.sync_copy(acc_s.at[pl.ds(s * SEG_PER_TILE, SEG_PER_TILE), :],
                    sum_out.at[c, pl.ds(s * SEG_PER_TILE, SEG_PER_TILE), :])
    pltpu.sync_copy(cnt_v, cnt_out.at[w])


_sc_segment_sums = functools.partial(
    pl.kernel,
    out_type=(
        jax.ShapeDtypeStruct((NC, S_PAD, D), jnp.float32),
        jax.ShapeDtypeStruct((NW, S_PAD), jnp.float32),
    ),
    mesh=plsc.VectorSubcoreMesh(
        core_axis_name="c", subcore_axis_name="s", num_cores=NC, num_subcores=NS
    ),
    compiler_params=pltpu.CompilerParams(needs_layout_passes=False),
    scratch_types=[
        pltpu.VMEM((3, B), jnp.int32),
        pltpu.VMEM((B, D), jnp.float32),
        pltpu.VMEM((B, D), jnp.float32),
        pltpu.VMEM((B, D), jnp.float32),
        pltpu.VMEM((S_PAD,), jnp.float32),
        pltpu.SemaphoreType.DMA,
        pltpu.SemaphoreType.DMA,
        pltpu.SemaphoreType.DMA,
        pltpu.SemaphoreType.DMA,
        pltpu.SemaphoreType.DMA,
        pltpu.SemaphoreType.DMA,
        pltpu.VMEM_SHARED((S_PAD, D), jnp.float32),
    ],
)(_sc_body)


def _combine_body(sum_ref, cnt_ref, out_ref):
    total = sum_ref[0] + sum_ref[1]
    cnt = jnp.sum(cnt_ref[...], axis=0)[:, None]
    out_ref[...] = jnp.where(cnt > 0, total / jnp.maximum(cnt, 1.0),
                             jnp.zeros_like(total))


_COMBINE_BS = 2048


def _combine(sums, cnts):
    grid = pl.cdiv(S, _COMBINE_BS)
    return pl.pallas_call(
        _combine_body,
        grid=(grid,),
        in_specs=[
            pl.BlockSpec((NC, _COMBINE_BS, D), lambda i: (0, i, 0)),
            pl.BlockSpec((NW, _COMBINE_BS), lambda i: (0, i)),
        ],
        out_specs=pl.BlockSpec((_COMBINE_BS, D), lambda i: (i, 0)),
        out_shape=jax.ShapeDtypeStruct((S, D), jnp.float32),
    )(sums, cnts)


def kernel(fine_scale_h_d, prolongation_map_fine_to_coarse):
    sums, cnts = _sc_segment_sums(fine_scale_h_d, prolongation_map_fine_to_coarse)
    return _combine(sums, cnts)


# speedup vs baseline: 15.7611x; 1.0077x over previous
import functools

import jax
import jax.numpy as jnp
from jax import lax
from jax.experimental import pallas as pl
from jax.experimental.pallas import tpu as pltpu
from jax.experimental.pallas import tpu_sc as plsc

N = 320000
D = 128
S = 10000
NC, NS = 2, 16
NW = NC * NS
ROWS_PER_W = N // NW
B = 80
CHUNKS = ROWS_PER_W // B
S_PAD = 10240
SEG_PER_TILE = S_PAD // NS
L = 16


def _sc_body(fine_hbm, ids_hbm,
             sum_out, cnt_out,
             idxs, buf0, buf1, buf2, cnt_v,
             sem_l0, sem_l1, sem_l2,
             sem_s0, sem_s1, sem_s2, acc_s):
    c = lax.axis_index("c")
    s = lax.axis_index("s")
    w = c * NS + s
    bufs = (buf0, buf1, buf2)
    sem_l = (sem_l0, sem_l1, sem_l2)
    sem_s = (sem_s0, sem_s1, sem_s2)

    def zero_buf(i, carry):
        buf0[i // (D // L), pl.ds((i % (D // L)) * L, L)] = jnp.zeros((L,), jnp.float32)
        return carry

    lax.fori_loop(0, B * D // L, zero_buf, 0)

    for q in range(SEG_PER_TILE // B):
        pltpu.async_copy(buf0, acc_s.at[pl.ds(s * SEG_PER_TILE + q * B, B), :],
                         sem_s0)

    def zero_cnt(j, carry):
        cnt_v[pl.ds(j * L, L)] = jnp.zeros((L,), jnp.float32)
        return carry

    lax.fori_loop(0, S_PAD // L, zero_cnt, 0)

    for q in range(SEG_PER_TILE // B):
        pltpu.make_async_copy(buf0, acc_s.at[pl.ds(s * SEG_PER_TILE + q * B, B), :],
                              sem_s0).wait()
    plsc.subcore_barrier()

    ones = jnp.ones((L,), jnp.float32)

    def row_slice(g):
        return fine_hbm.at[pl.ds(w * ROWS_PER_W + g * B, B), :]

    def id_slice(g):
        return ids_hbm.at[pl.ds(w * ROWS_PER_W + g * B, B)]

    def fire_loads(g, b):
        pltpu.async_copy(row_slice(g), bufs[b], sem_l[b])
        pltpu.async_copy(id_slice(g), idxs.at[b], sem_l[b])

    def wait_loads(g, b):
        pltpu.make_async_copy(row_slice(g), bufs[b], sem_l[b]).wait()
        pltpu.make_async_copy(id_slice(g), idxs.at[b], sem_l[b]).wait()

    def counts(b):
        for j in range(B // L):
            plsc.addupdate_scatter(cnt_v, [idxs[b, pl.ds(j * L, L)]], ones)

    def stage(g, b):
        bp = (b + 2) % 3
        wait_loads(g, b)
        pltpu.async_copy(bufs[b], acc_s.at[idxs.at[b]], sem_s[b], add=True)
        counts(b)
        pltpu.make_async_copy(bufs[bp], acc_s.at[idxs.at[bp]], sem_s[bp]).wait()
        fire_loads(g + 2, bp)

    fire_loads(0, 0)
    fire_loads(1, 1)
    wait_loads(0, 0)
    pltpu.async_copy(buf0, acc_s.at[idxs.at[0]], sem_s0, add=True)
    counts(0)
    fire_loads(2, 2)

    def body(k, carry):
        g0 = 3 * k + 1
        stage(g0, 1)
        stage(g0 + 1, 2)
        stage(g0 + 2, 0)
        return carry

    lax.fori_loop(0, (CHUNKS - 5) // 3, body, 0)

    stage(CHUNKS - 4, 1)
    stage(CHUNKS - 3, 2)

    g = CHUNKS - 2
    wait_loads(g, 0)
    pltpu.async_copy(buf0, acc_s.at[idxs.at[0]], sem_s0, add=True)
    counts(0)
    pltpu.make_async_copy(buf2, acc_s.at[idxs.at[2]], sem_s2).wait()
    g = CHUNKS - 1
    wait_loads(g, 1)
    pltpu.async_copy(buf1, acc_s.at[idxs.at[1]], sem_s1, add=True)
    counts(1)
    pltpu.make_async_copy(buf0, acc_s.at[idxs.at[0]], sem_s0).wait()
    pltpu.make_async_copy(buf1, acc_s.at[idxs.at[1]], sem_s1).wait()

    pltpu.async_copy(cnt_v, cnt_out.at[w], sem_l0)
    plsc.subcore_barrier()
    pltpu.sync_copy(acc_s.at[pl.ds(s * SEG_PER_TILE, SEG_PER_TILE), :],
                    sum_out.at[c, pl.ds(s * SEG_PER_TILE, SEG_PER_TILE), :])
    pltpu.make_async_copy(cnt_v, cnt_out.at[w], sem_l0).wait()


_sc_segment_sums = functools.partial(
    pl.kernel,
    out_type=(
        jax.ShapeDtypeStruct((NC, S_PAD, D), jnp.float32),
        jax.ShapeDtypeStruct((NW, S_PAD), jnp.float32),
    ),
    mesh=plsc.VectorSubcoreMesh(
        core_axis_name="c", subcore_axis_name="s", num_cores=NC, num_subcores=NS
    ),
    compiler_params=pltpu.CompilerParams(needs_layout_passes=False),
    scratch_types=[
        pltpu.VMEM((3, B), jnp.int32),
        pltpu.VMEM((B, D), jnp.float32),
        pltpu.VMEM((B, D), jnp.float32),
        pltpu.VMEM((B, D), jnp.float32),
        pltpu.VMEM((S_PAD,), jnp.float32),
        pltpu.SemaphoreType.DMA,
        pltpu.SemaphoreType.DMA,
        pltpu.SemaphoreType.DMA,
        pltpu.SemaphoreType.DMA,
        pltpu.SemaphoreType.DMA,
        pltpu.SemaphoreType.DMA,
        pltpu.VMEM_SHARED((S_PAD, D), jnp.float32),
    ],
)(_sc_body)


def _combine_body(sum_ref, cnt_ref, out_ref):
    total = sum_ref[0, :S, :] + sum_ref[1, :S, :]
    cnt = jnp.sum(cnt_ref[:, :S], axis=0)[:, None]
    out_ref[...] = jnp.where(cnt > 0, total / jnp.maximum(cnt, 1.0),
                             jnp.zeros_like(total))


_COMBINE_BS = 2048


def _combine(sums, cnts):
    return pl.pallas_call(
        _combine_body,
        out_shape=jax.ShapeDtypeStruct((S, D), jnp.float32),
    )(sums, cnts)


def kernel(fine_scale_h_d, prolongation_map_fine_to_coarse):
    sums, cnts = _sc_segment_sums(fine_scale_h_d, prolongation_map_fine_to_coarse)
    return _combine(sums, cnts)
